# baseline (speedup 1.0000x reference)
import jax
import jax.numpy as jnp
from jax.experimental import pallas as pl
from jax.experimental.pallas import tpu as pltpu

D = 512
K = 1024
Q = 12
B = 512
CHUNK = 128


def _prep_kernel(cb_ref, cc_ref, cb3_ref):
    cb = cb_ref[...]
    cc_ref[...] = jnp.sum(cb * cb, axis=-1, keepdims=True).transpose(0, 2, 1)
    hi = cb.astype(jnp.bfloat16)
    e1 = cb - hi.astype(jnp.float32)
    mid = e1.astype(jnp.bfloat16)
    lo = (e1 - mid.astype(jnp.float32)).astype(jnp.bfloat16)
    cb3_ref[...] = jnp.concatenate([hi, mid, lo], axis=-1)


def _rvq_kernel(x_ref, cb3_ref, cc_ref, out_ref):
    r = x_ref[...]
    nrows = r.shape[0]
    iota = jax.lax.broadcasted_iota(jnp.int32, (nrows, K), 1)
    cols = []
    for q in range(Q):
        rr = jnp.sum(r * r, axis=1, keepdims=True)
        rbf = (r + r).astype(jnp.bfloat16)
        acc_v = acc_i = None
        for m in range(K // CHUNK):
            lo_k, hi_k = m * CHUNK, (m + 1) * CHUNK
            s2m = jax.lax.dot_general(
                rbf, cb3_ref[q][lo_k:hi_k, :D],
                (((1,), (1,)), ((), ())),
                preferred_element_type=jnp.float32)
            dm = rr - s2m + cc_ref[q][:, lo_k:hi_k]
            im = iota[:, lo_k:hi_k]
            if acc_v is None:
                acc_v, acc_i = dm, im
            else:
                take = dm < acc_v
                acc_v = jnp.where(take, dm, acc_v)
                acc_i = jnp.where(take, im, acc_i)
        mn = jnp.min(acc_v, axis=1, keepdims=True)
        idx = jnp.min(jnp.where(acc_v == mn, acc_i, K), axis=1)
        cols.append(idx)
        if q < Q - 1:
            oh = (iota == idx[:, None]).astype(jnp.bfloat16)
            g = jax.lax.dot_general(
                oh, cb3_ref[q], (((1,), (0,)), ((), ())),
                preferred_element_type=jnp.float32)
            quant = g[:, :D] + g[:, D:2 * D] + g[:, 2 * D:]
            r = r - quant
    out_ref[...] = jnp.stack(cols, axis=-1)


def kernel(embedding, codebooks):
    n = embedding.shape[0]
    cc, cb3 = pl.pallas_call(
        _prep_kernel,
        grid=(Q,),
        in_specs=[pl.BlockSpec((1, K, D), lambda q: (q, 0, 0))],
        out_specs=[
            pl.BlockSpec((1, 1, K), lambda q: (q, 0, 0)),
            pl.BlockSpec((1, K, 3 * D), lambda q: (q, 0, 0)),
        ],
        out_shape=[
            jax.ShapeDtypeStruct((Q, 1, K), jnp.float32),
            jax.ShapeDtypeStruct((Q, K, 3 * D), jnp.bfloat16),
        ],
    )(codebooks)
    out = pl.pallas_call(
        _rvq_kernel,
        grid=(n // B,),
        in_specs=[
            pl.BlockSpec((B, D), lambda i: (i, 0)),
            pl.BlockSpec((Q, K, 3 * D), lambda i: (0, 0, 0)),
            pl.BlockSpec((Q, 1, K), lambda i: (0, 0, 0)),
        ],
        out_specs=pl.BlockSpec((B, Q), lambda i: (i, 0)),
        out_shape=jax.ShapeDtypeStruct((n, Q), jnp.int32),
        compiler_params=pltpu.CompilerParams(
            dimension_semantics=("parallel",)),
    )(embedding, cb3, cc)
    return out[:, :, None]


# speedup vs baseline: 1.1408x; 1.1408x over previous
import jax
import jax.numpy as jnp
from jax.experimental import pallas as pl
from jax.experimental.pallas import tpu as pltpu

D = 512
K = 1024
Q = 12
B = 512
CHUNK = 256


def _prep_kernel(cb_ref, cc_ref, cb3_ref):
    cb = cb_ref[...]
    cc_ref[...] = jnp.sum(cb * cb, axis=-1, keepdims=True).transpose(0, 2, 1)
    hi = cb.astype(jnp.bfloat16)
    e1 = cb - hi.astype(jnp.float32)
    mid = e1.astype(jnp.bfloat16)
    lo = (e1 - mid.astype(jnp.float32)).astype(jnp.bfloat16)
    cb3_ref[...] = jnp.concatenate([hi, mid, lo], axis=-1)


def _rvq_kernel(x_ref, cb3_ref, cc_ref, out_ref):
    r = x_ref[...]
    nrows = r.shape[0]
    iota = jax.lax.broadcasted_iota(jnp.int32, (nrows, K), 1)
    cols = []
    for q in range(Q):
        rr = jnp.sum(r * r, axis=1, keepdims=True)
        rbf = (r + r).astype(jnp.bfloat16)
        acc_v = acc_i = None
        for m in range(K // CHUNK):
            lo_k, hi_k = m * CHUNK, (m + 1) * CHUNK
            s2m = jax.lax.dot_general(
                rbf, cb3_ref[q][lo_k:hi_k, :D],
                (((1,), (1,)), ((), ())),
                preferred_element_type=jnp.float32)
            dm = rr - s2m + cc_ref[q][:, lo_k:hi_k]
            im = iota[:, lo_k:hi_k]
            if acc_v is None:
                acc_v, acc_i = dm, im
            else:
                take = dm < acc_v
                acc_v = jnp.where(take, dm, acc_v)
                acc_i = jnp.where(take, im, acc_i)
        mn = jnp.min(acc_v, axis=1, keepdims=True)
        idx = jnp.min(jnp.where(acc_v == mn, acc_i, K), axis=1)
        cols.append(idx)
        if q < Q - 1:
            oh = (iota == idx[:, None]).astype(jnp.bfloat16)
            g = jax.lax.dot_general(
                oh, cb3_ref[q], (((1,), (0,)), ((), ())),
                preferred_element_type=jnp.float32)
            quant = g[:, :D] + g[:, D:2 * D] + g[:, 2 * D:]
            r = r - quant
    out_ref[...] = jnp.stack(cols, axis=-1)


def kernel(embedding, codebooks):
    n = embedding.shape[0]
    cc, cb3 = pl.pallas_call(
        _prep_kernel,
        grid=(Q,),
        in_specs=[pl.BlockSpec((1, K, D), lambda q: (q, 0, 0))],
        out_specs=[
            pl.BlockSpec((1, 1, K), lambda q: (q, 0, 0)),
            pl.BlockSpec((1, K, 3 * D), lambda q: (q, 0, 0)),
        ],
        out_shape=[
            jax.ShapeDtypeStruct((Q, 1, K), jnp.float32),
            jax.ShapeDtypeStruct((Q, K, 3 * D), jnp.bfloat16),
        ],
    )(codebooks)
    out = pl.pallas_call(
        _rvq_kernel,
        grid=(n // B,),
        in_specs=[
            pl.BlockSpec((B, D), lambda i: (i, 0)),
            pl.BlockSpec((Q, K, 3 * D), lambda i: (0, 0, 0)),
            pl.BlockSpec((Q, 1, K), lambda i: (0, 0, 0)),
        ],
        out_specs=pl.BlockSpec((B, Q), lambda i: (i, 0)),
        out_shape=jax.ShapeDtypeStruct((n, Q), jnp.int32),
        compiler_params=pltpu.CompilerParams(
            dimension_semantics=("parallel",)),
    )(embedding, cb3, cc)
    return out[:, :, None]


# speedup vs baseline: 1.1428x; 1.0017x over previous
import jax
import jax.numpy as jnp
from jax.experimental import pallas as pl
from jax.experimental.pallas import tpu as pltpu

D = 512
K = 1024
Q = 12
B = 512
CHUNK = 256


def _prep_kernel(cb_ref, cc_ref, cb3_ref):
    cb = cb_ref[...]
    cc_ref[...] = jnp.sum(cb * cb, axis=-1, keepdims=True).transpose(0, 2, 1)
    hi = cb.astype(jnp.bfloat16)
    e1 = cb - hi.astype(jnp.float32)
    mid = e1.astype(jnp.bfloat16)
    lo = (e1 - mid.astype(jnp.float32)).astype(jnp.bfloat16)
    cb3_ref[...] = jnp.concatenate([hi, mid, lo], axis=-1)


def _rvq_kernel(x_ref, cb3_ref, cc_ref, out_ref):
    r = x_ref[...]
    nrows = r.shape[0]
    iota = jax.lax.broadcasted_iota(jnp.int32, (nrows, K), 1)
    cols = []
    for q in range(Q):
        rr = jnp.sum(r * r, axis=1, keepdims=True)
        rbf = (r + r).astype(jnp.bfloat16)
        acc_v = acc_i = None
        for m in range(K // CHUNK):
            lo_k, hi_k = m * CHUNK, (m + 1) * CHUNK
            s2m = jax.lax.dot_general(
                rbf, cb3_ref[q][lo_k:hi_k, :D],
                (((1,), (1,)), ((), ())),
                preferred_element_type=jnp.float32)
            dm = rr - s2m + cc_ref[q][:, lo_k:hi_k]
            im = iota[:, lo_k:hi_k]
            if acc_v is None:
                acc_v, acc_i = dm, im
            else:
                take = dm < acc_v
                acc_v = jnp.where(take, dm, acc_v)
                acc_i = jnp.where(take, im, acc_i)
        mn = jnp.min(acc_v, axis=1, keepdims=True)
        idx = jnp.min(jnp.where(acc_v == mn, acc_i, K), axis=1)
        cols.append(idx)
        if q < Q - 1:
            oh = (iota == idx[:, None]).astype(jnp.bfloat16)
            g = jax.lax.dot_general(
                oh, cb3_ref[q], (((1,), (0,)), ((), ())),
                preferred_element_type=jnp.float32)
            quant = g[:, :D] + g[:, D:2 * D] + g[:, 2 * D:]
            r = r - quant
    out_ref[...] = jnp.stack(cols, axis=-1)


def kernel(embedding, codebooks):
    n = embedding.shape[0]
    cc, cb3 = pl.pallas_call(
        _prep_kernel,
        grid=(Q,),
        in_specs=[pl.BlockSpec((1, K, D), lambda q: (q, 0, 0))],
        out_specs=[
            pl.BlockSpec((1, 1, K), lambda q: (q, 0, 0)),
            pl.BlockSpec((1, K, 3 * D), lambda q: (q, 0, 0)),
        ],
        out_shape=[
            jax.ShapeDtypeStruct((Q, 1, K), jnp.float32),
            jax.ShapeDtypeStruct((Q, K, 3 * D), jnp.bfloat16),
        ],
    )(codebooks)
    out = pl.pallas_call(
        _rvq_kernel,
        grid=(n // B,),
        in_specs=[
            pl.BlockSpec((B, D), lambda i: (i, 0)),
            pl.BlockSpec((Q, K, 3 * D), lambda i: (0, 0, 0)),
            pl.BlockSpec((Q, 1, K), lambda i: (0, 0, 0)),
        ],
        out_specs=pl.BlockSpec((B, Q), lambda i: (i, 0)),
        out_shape=jax.ShapeDtypeStruct((n, Q), jnp.int32),
        compiler_params=pltpu.CompilerParams(
            dimension_semantics=("arbitrary",)),
    )(embedding, cb3, cc)
    return out[:, :, None]
